# Initial kernel scaffold; baseline (speedup 1.0000x reference)
import jax
import jax.numpy as jnp
from jax.experimental import pallas as pl


def kernel(x, edge_index, batch, W1_0, b1_0, W2_0, b2_0, g_0, be_0, W1_1, b1_1, W2_1, b2_1, g_1, be_1, W1_2, b1_2, W2_2, b2_2, g_2, be_2, Wh1, bh1, Wh2, bh2):
    raise NotImplementedError("write your pallas kernel here")


# speedup vs baseline: 5.3984x; 5.3984x over previous
import functools

import jax
import jax.numpy as jnp
from jax import lax
from jax.experimental import pallas as pl
from jax.experimental.pallas import tpu as pltpu
from jax.experimental.pallas import tpu_sc as plsc

_NC = 2
_NS = 16
_NW = _NC * _NS
_CHUNK = 128
_G = 128
_LANES = 16

_DOT = dict(preferred_element_type=jnp.float32, precision=lax.Precision.HIGHEST)


def _mm(a, b):
    return lax.dot_general(a, b, (((1,), (0,)), ((), ())), **_DOT)


def _sc_aggregate(p_pad, srcs, dsts):
    n_pad, h = p_pad.shape
    cpw = srcs.shape[1]
    rows_per_tile = n_pad // _NS
    nfull = rows_per_tile // _CHUNK
    rem = rows_per_tile % _CHUNK
    mesh = plsc.VectorSubcoreMesh(core_axis_name="c", subcore_axis_name="s")

    def body(p_hbm, src_hbm, dst_hbm, out0, out1,
             src_v, dst_v, zbuf, rows0, rows1, acc, sem0, sem1):
        cid = lax.axis_index("c")
        sid = lax.axis_index("s")
        wid = sid * _NC + cid
        r0 = sid * rows_per_tile

        zv = jnp.zeros((_LANES,), jnp.float32)

        def zrow(r, carry):
            for c in range(h // _LANES):
                zbuf[r, pl.ds(c * _LANES, _LANES)] = zv
            return carry

        lax.fori_loop(0, _CHUNK, zrow, 0)
        for k in range(nfull):
            pltpu.sync_copy(zbuf, acc.at[pl.ds(r0 + k * _CHUNK, _CHUNK)])
        if rem:
            pltpu.sync_copy(zbuf.at[pl.ds(0, rem)],
                            acc.at[pl.ds(r0 + nfull * _CHUNK, rem)])

        pltpu.sync_copy(src_hbm.at[wid], src_v)
        pltpu.sync_copy(dst_hbm.at[wid], dst_v)

        plsc.subcore_barrier()

        pltpu.async_copy(p_hbm.at[src_v.at[0]], rows0, sem0)
        pltpu.async_copy(p_hbm.at[src_v.at[1]], rows1, sem1)

        def step(j, carry):
            jj = 2 * j
            pltpu.make_async_copy(p_hbm.at[src_v.at[jj]], rows0, sem0).wait()
            pltpu.sync_copy(rows0, acc.at[dst_v.at[jj]], add=True)

            @pl.when(j < cpw // 2 - 1)
            def _():
                pltpu.async_copy(p_hbm.at[src_v.at[jj + 2]], rows0, sem0)

            pltpu.make_async_copy(p_hbm.at[src_v.at[jj + 1]], rows1, sem1).wait()
            pltpu.sync_copy(rows1, acc.at[dst_v.at[jj + 1]], add=True)

            @pl.when(j < cpw // 2 - 1)
            def _():
                pltpu.async_copy(p_hbm.at[src_v.at[jj + 3]], rows1, sem1)

            return carry

        lax.fori_loop(0, cpw // 2, step, 0)

        plsc.subcore_barrier()

        @pl.when(cid == 0)
        def _():
            pltpu.sync_copy(acc.at[pl.ds(r0, rows_per_tile)],
                            out0.at[pl.ds(r0, rows_per_tile)])

        @pl.when(cid == 1)
        def _():
            pltpu.sync_copy(acc.at[pl.ds(r0, rows_per_tile)],
                            out1.at[pl.ds(r0, rows_per_tile)])

    fn = pl.kernel(
        body,
        out_type=(jax.ShapeDtypeStruct((n_pad, h), jnp.float32),
                  jax.ShapeDtypeStruct((n_pad, h), jnp.float32)),
        mesh=mesh,
        scratch_types=[
            pltpu.VMEM((cpw, _CHUNK), jnp.int32),
            pltpu.VMEM((cpw, _CHUNK), jnp.int32),
            pltpu.VMEM((_CHUNK, h), jnp.float32),
            pltpu.VMEM((_CHUNK, h), jnp.float32),
            pltpu.VMEM((_CHUNK, h), jnp.float32),
            pltpu.VMEM_SHARED((n_pad, h), jnp.float32),
            pltpu.SemaphoreType.DMA,
            pltpu.SemaphoreType.DMA,
        ],
        compiler_params=pltpu.CompilerParams(use_tc_tiling_on_sc=False),
    )
    return fn(p_pad, srcs, dsts)


def _proj_body(x_ref, w_ref, o_ref):
    o_ref[...] = _mm(x_ref[...], w_ref[...])


def _layer_post(n, p_ref, a0_ref, a1_ref, b1_ref, w2_ref, b2_ref, g_ref, be_ref):
    n_pad, h = p_ref.shape
    u = p_ref[...] + a0_ref[...] + a1_ref[...] + b1_ref[...]
    u = jnp.maximum(u, 0.0)
    v = _mm(u, w2_ref[...]) + b2_ref[...]
    rmask = (lax.broadcasted_iota(jnp.int32, (n_pad, h), 0) < n).astype(jnp.float32)
    vm = v * rmask
    s1 = jnp.sum(vm, axis=0, keepdims=True)
    s2 = jnp.sum(vm * vm, axis=0, keepdims=True)
    mean = s1 / n
    var = s2 / n - mean * mean
    inv = lax.rsqrt(var + 1e-5)
    hh = jnp.maximum((v - mean) * inv * g_ref[...] + be_ref[...], 0.0)
    return hh * rmask


def _mid_body(n, p_ref, a0_ref, a1_ref, b1_ref, w2_ref, b2_ref, g_ref, be_ref,
              w1n_ref, o_ref):
    hh = _layer_post(n, p_ref, a0_ref, a1_ref, b1_ref, w2_ref, b2_ref, g_ref,
                     be_ref)
    o_ref[...] = _mm(hh, w1n_ref[...])


def _fin_body(n, p_ref, a0_ref, a1_ref, b1_ref, w2_ref, b2_ref, g_ref, be_ref,
              batch_ref, wh1_ref, bh1_ref, wh2_ref, bh2_ref, o_ref):
    hh = _layer_post(n, p_ref, a0_ref, a1_ref, b1_ref, w2_ref, b2_ref, g_ref,
                     be_ref)
    n_pad = p_ref.shape[0]
    onehot = (lax.broadcasted_iota(jnp.int32, (_G, n_pad), 0)
              == batch_ref[...]).astype(jnp.float32)
    hg = _mm(onehot, hh)
    t = jnp.maximum(_mm(hg, wh1_ref[...]) + bh1_ref[...], 0.0)
    o_ref[...] = _mm(t, wh2_ref[...]) + bh2_ref[...]


def kernel(x, edge_index, batch,
           W1_0, b1_0, W2_0, b2_0, g_0, be_0,
           W1_1, b1_1, W2_1, b2_1, g_1, be_1,
           W1_2, b1_2, W2_2, b2_2, g_2, be_2,
           Wh1, bh1, Wh2, bh2):
    n, d = x.shape
    h = W1_0.shape[1]
    e = edge_index.shape[1]
    c = Wh2.shape[1]

    n_pad = -(-(n + 1) // (_NS * 8)) * (_NS * 8)
    cpw = -(-e // (_NW * _CHUNK))
    cpw += cpw % 2
    e_pad = _NW * cpw * _CHUNK

    pad = jnp.full((e_pad - e,), n, jnp.int32)
    srcs = jnp.concatenate([edge_index[0], pad]).reshape(_NW, cpw, _CHUNK)
    dsts = jnp.concatenate([edge_index[1], pad]).reshape(_NW, cpw, _CHUNK)
    x_pad = jnp.pad(x, ((0, n_pad - n), (0, 0)))
    batch_pad = jnp.pad(batch, (0, n_pad - n),
                        constant_values=-1).reshape(1, n_pad)

    row = lambda a: a.reshape(1, -1)
    ws = {
        0: (row(b1_0), W2_0, row(b2_0), row(g_0), row(be_0)),
        1: (row(b1_1), W2_1, row(b2_1), row(g_1), row(be_1)),
        2: (row(b1_2), W2_2, row(b2_2), row(g_2), row(be_2)),
    }

    p = pl.pallas_call(
        _proj_body,
        out_shape=jax.ShapeDtypeStruct((n_pad, h), jnp.float32),
    )(x_pad, W1_0)

    for i in range(3):
        a0, a1 = _sc_aggregate(p, srcs, dsts)
        b1r, W2, b2r, gr, ber = ws[i]
        if i < 2:
            w1n = W1_1 if i == 0 else W1_2
            p = pl.pallas_call(
                functools.partial(_mid_body, n),
                out_shape=jax.ShapeDtypeStruct((n_pad, h), jnp.float32),
            )(p, a0, a1, b1r, W2, b2r, gr, ber, w1n)
        else:
            out = pl.pallas_call(
                functools.partial(_fin_body, n),
                out_shape=jax.ShapeDtypeStruct((_G, c), jnp.float32),
            )(p, a0, a1, b1r, W2, b2r, gr, ber, batch_pad,
              Wh1, row(bh1), Wh2, row(bh2))
    return out


# speedup vs baseline: 5.7300x; 1.0614x over previous
import functools

import jax
import jax.numpy as jnp
from jax import lax
from jax.experimental import pallas as pl
from jax.experimental.pallas import tpu as pltpu
from jax.experimental.pallas import tpu_sc as plsc

_NC = 2
_NS = 16
_NW = _NC * _NS
_CHUNK = 128
_NBUF = 4
_G = 128
_LANES = 16

_DOT = dict(preferred_element_type=jnp.float32, precision=lax.Precision.HIGHEST)


def _mm(a, b):
    return lax.dot_general(a, b, (((1,), (0,)), ((), ())), **_DOT)


def _sc_aggregate(p_pad, srcs, dsts):
    n_pad, h = p_pad.shape
    cpw = srcs.shape[1]
    rows_per_tile = n_pad // _NS
    nfull = rows_per_tile // _CHUNK
    rem = rows_per_tile % _CHUNK
    mesh = plsc.VectorSubcoreMesh(core_axis_name="c", subcore_axis_name="s")

    def body(p_hbm, src_hbm, dst_hbm, out0, out1,
             src_v, dst_v, zbuf, rows0, rows1, rows2, rows3, acc,
             sem0, sem1, sem2, sem3):
        cid = lax.axis_index("c")
        sid = lax.axis_index("s")
        wid = sid * _NC + cid
        r0 = sid * rows_per_tile

        zv = jnp.zeros((_LANES,), jnp.float32)

        def zrow(r, carry):
            for c in range(h // _LANES):
                zbuf[r, pl.ds(c * _LANES, _LANES)] = zv
            return carry

        lax.fori_loop(0, _CHUNK, zrow, 0)
        for k in range(nfull):
            pltpu.sync_copy(zbuf, acc.at[pl.ds(r0 + k * _CHUNK, _CHUNK)])
        if rem:
            pltpu.sync_copy(zbuf.at[pl.ds(0, rem)],
                            acc.at[pl.ds(r0 + nfull * _CHUNK, rem)])

        pltpu.sync_copy(src_hbm.at[wid], src_v)
        pltpu.sync_copy(dst_hbm.at[wid], dst_v)

        plsc.subcore_barrier()

        rows = (rows0, rows1, rows2, rows3)
        sems = (sem0, sem1, sem2, sem3)
        for b in range(_NBUF):
            pltpu.async_copy(p_hbm.at[src_v.at[b]], rows[b], sems[b])

        def step(t, carry):
            j = t * _NBUF
            for b in range(_NBUF):
                jj = j + b
                pltpu.make_async_copy(p_hbm.at[src_v.at[jj]], rows[b],
                                      sems[b]).wait()
                pltpu.sync_copy(rows[b], acc.at[dst_v.at[jj]], add=True)

                @pl.when(jj + _NBUF < cpw)
                def _():
                    pltpu.async_copy(p_hbm.at[src_v.at[jj + _NBUF]], rows[b],
                                     sems[b])

            return carry

        lax.fori_loop(0, cpw // _NBUF, step, 0)

        plsc.subcore_barrier()

        @pl.when(cid == 0)
        def _():
            pltpu.sync_copy(acc.at[pl.ds(r0, rows_per_tile)],
                            out0.at[pl.ds(r0, rows_per_tile)])

        @pl.when(cid == 1)
        def _():
            pltpu.sync_copy(acc.at[pl.ds(r0, rows_per_tile)],
                            out1.at[pl.ds(r0, rows_per_tile)])

    fn = pl.kernel(
        body,
        out_type=(jax.ShapeDtypeStruct((n_pad, h), jnp.float32),
                  jax.ShapeDtypeStruct((n_pad, h), jnp.float32)),
        mesh=mesh,
        scratch_types=[
            pltpu.VMEM((cpw, _CHUNK), jnp.int32),
            pltpu.VMEM((cpw, _CHUNK), jnp.int32),
            pltpu.VMEM((_CHUNK, h), jnp.float32),
            pltpu.VMEM((_CHUNK, h), jnp.float32),
            pltpu.VMEM((_CHUNK, h), jnp.float32),
            pltpu.VMEM((_CHUNK, h), jnp.float32),
            pltpu.VMEM((_CHUNK, h), jnp.float32),
            pltpu.VMEM_SHARED((n_pad, h), jnp.float32),
            pltpu.SemaphoreType.DMA,
            pltpu.SemaphoreType.DMA,
            pltpu.SemaphoreType.DMA,
            pltpu.SemaphoreType.DMA,
        ],
        compiler_params=pltpu.CompilerParams(use_tc_tiling_on_sc=False),
    )
    return fn(p_pad, srcs, dsts)


def _proj_body(x_ref, w_ref, o_ref):
    o_ref[...] = _mm(x_ref[...], w_ref[...])


def _layer_post(n, p_ref, a0_ref, a1_ref, b1_ref, w2_ref, b2_ref, g_ref, be_ref):
    n_pad, h = p_ref.shape
    u = p_ref[...] + a0_ref[...] + a1_ref[...] + b1_ref[...]
    u = jnp.maximum(u, 0.0)
    v = _mm(u, w2_ref[...]) + b2_ref[...]
    rmask = (lax.broadcasted_iota(jnp.int32, (n_pad, h), 0) < n).astype(jnp.float32)
    vm = v * rmask
    s1 = jnp.sum(vm, axis=0, keepdims=True)
    s2 = jnp.sum(vm * vm, axis=0, keepdims=True)
    mean = s1 / n
    var = s2 / n - mean * mean
    inv = lax.rsqrt(var + 1e-5)
    hh = jnp.maximum((v - mean) * inv * g_ref[...] + be_ref[...], 0.0)
    return hh * rmask


def _mid_body(n, p_ref, a0_ref, a1_ref, b1_ref, w2_ref, b2_ref, g_ref, be_ref,
              w1n_ref, o_ref):
    hh = _layer_post(n, p_ref, a0_ref, a1_ref, b1_ref, w2_ref, b2_ref, g_ref,
                     be_ref)
    o_ref[...] = _mm(hh, w1n_ref[...])


def _fin_body(n, p_ref, a0_ref, a1_ref, b1_ref, w2_ref, b2_ref, g_ref, be_ref,
              batch_ref, wh1_ref, bh1_ref, wh2_ref, bh2_ref, o_ref):
    hh = _layer_post(n, p_ref, a0_ref, a1_ref, b1_ref, w2_ref, b2_ref, g_ref,
                     be_ref)
    n_pad = p_ref.shape[0]
    onehot = (lax.broadcasted_iota(jnp.int32, (_G, n_pad), 0)
              == batch_ref[...]).astype(jnp.float32)
    hg = _mm(onehot, hh)
    t = jnp.maximum(_mm(hg, wh1_ref[...]) + bh1_ref[...], 0.0)
    o_ref[...] = _mm(t, wh2_ref[...]) + bh2_ref[...]


def kernel(x, edge_index, batch,
           W1_0, b1_0, W2_0, b2_0, g_0, be_0,
           W1_1, b1_1, W2_1, b2_1, g_1, be_1,
           W1_2, b1_2, W2_2, b2_2, g_2, be_2,
           Wh1, bh1, Wh2, bh2):
    n, d = x.shape
    h = W1_0.shape[1]
    e = edge_index.shape[1]
    c = Wh2.shape[1]

    n_pad = -(-(n + 1) // (_NS * 8)) * (_NS * 8)
    cpw = -(-(-(-e // (_NW * _CHUNK))) // _NBUF) * _NBUF
    e_pad = _NW * cpw * _CHUNK

    npe = e_pad - e
    src_pad = jnp.full((npe,), n, jnp.int32)
    dst_pad = (jnp.arange(npe, dtype=jnp.int32) % (n_pad - n)) + n
    shard = lambda a: a.reshape(cpw, _CHUNK, _NW).transpose(2, 0, 1)
    srcs = shard(jnp.concatenate([edge_index[0], src_pad]))
    dsts = shard(jnp.concatenate([edge_index[1], dst_pad]))
    x_pad = jnp.pad(x, ((0, n_pad - n), (0, 0)))
    batch_pad = jnp.pad(batch, (0, n_pad - n),
                        constant_values=-1).reshape(1, n_pad)

    row = lambda a: a.reshape(1, -1)
    ws = {
        0: (row(b1_0), W2_0, row(b2_0), row(g_0), row(be_0)),
        1: (row(b1_1), W2_1, row(b2_1), row(g_1), row(be_1)),
        2: (row(b1_2), W2_2, row(b2_2), row(g_2), row(be_2)),
    }

    p = pl.pallas_call(
        _proj_body,
        out_shape=jax.ShapeDtypeStruct((n_pad, h), jnp.float32),
    )(x_pad, W1_0)

    for i in range(3):
        a0, a1 = _sc_aggregate(p, srcs, dsts)
        b1r, W2, b2r, gr, ber = ws[i]
        if i < 2:
            w1n = W1_1 if i == 0 else W1_2
            p = pl.pallas_call(
                functools.partial(_mid_body, n),
                out_shape=jax.ShapeDtypeStruct((n_pad, h), jnp.float32),
            )(p, a0, a1, b1r, W2, b2r, gr, ber, w1n)
        else:
            out = pl.pallas_call(
                functools.partial(_fin_body, n),
                out_shape=jax.ShapeDtypeStruct((_G, c), jnp.float32),
            )(p, a0, a1, b1r, W2, b2r, gr, ber, batch_pad,
              Wh1, row(bh1), Wh2, row(bh2))
    return out
